# baseline (speedup 1.0000x reference)
import jax
import jax.numpy as jnp
from jax.experimental import pallas as pl
from jax.experimental.pallas import tpu as pltpu

_B = 64; _K = 256; _NN = 64; _OBS = 8; _PRED = 12; _INS = 2
_E = 64; _H = 4; _FF = 128
_DH = _E // _H
_PK = 100
_NK = 20
_DIN = _OBS * _INS
_DMODE = _PRED * 2
_GRP = 4

_PARAM_ORDER = (
    'W_emb', 'b_emb',
    'enc_Wq', 'enc_bq', 'enc_Wk', 'enc_bk', 'enc_Wv', 'enc_bv',
    'enc_Wo', 'enc_bo', 'enc_ln1_g', 'enc_ln1_b',
    'enc_W1', 'enc_b1', 'enc_W2', 'enc_b2', 'enc_ln2_g', 'enc_ln2_b',
    'dec_Wq', 'dec_bq', 'dec_Wk', 'dec_bk', 'dec_Wv', 'dec_bv',
    'dec_Wo', 'dec_bo', 'dec_ln1_g', 'dec_ln1_b',
    'dec_W1', 'dec_b1', 'dec_W2', 'dec_b2', 'dec_ln2_g', 'dec_ln2_b',
    'W_cls', 'b_cls', 'W_cls2', 'b_cls2', 'W_nei', 'b_nei',
    'W_reg', 'b_reg',
)

_PREC = jax.lax.Precision.HIGHEST


def _dot(a, b):
    return jax.lax.dot_general(a, b, (((1,), (0,)), ((), ())),
                               preferred_element_type=jnp.float32)


def _dot_x(a, b):
    return jax.lax.dot_general(a, b, (((1,), (0,)), ((), ())),
                               preferred_element_type=jnp.float32,
                               precision=_PREC)


def _dot_t(a, b):
    return jax.lax.dot_general(a, b, (((1,), (1,)), ((), ())),
                               preferred_element_type=jnp.float32)


def _col_to_row(v, ident):
    return jax.lax.dot_general(v, ident, (((0,), (0,)), ((), ())),
                               preferred_element_type=jnp.float32,
                               precision=_PREC)


def _row_to_col(v, ident):
    return jax.lax.dot_general(ident, v, (((1,), (1,)), ((), ())),
                               preferred_element_type=jnp.float32,
                               precision=_PREC)


def _chunk_sum(x):
    w = x.shape[-1]
    if w % 8:
        pad = 8 - w % 8
        x = jnp.concatenate(
            [x, jnp.zeros((x.shape[0], pad), x.dtype)], axis=1)
        w += pad
    acc = x[:, 0:8]
    for i in range(1, w // 8):
        acc = acc + x[:, 8 * i:8 * i + 8]
    acc = acc[:, 0:4] + acc[:, 4:8]
    acc = acc[:, 0:2] + acc[:, 2:4]
    return acc[:, 0:1] + acc[:, 1:2]


def _ln(x, g, b):
    n = float(x.shape[-1])
    m = _chunk_sum(x) / n
    d = x - m
    v = _chunk_sum(d * d) / n
    return d / jnp.sqrt(v + 1e-5) * g + b


def _attn_enc(q, k, v, heads):
    outs = []
    for h in range(heads):
        sl = slice(h * _DH, (h + 1) * _DH)
        qh, kh, vh = q[:, sl], k[:, sl], v[:, sl]
        s_l = _dot_t(qh, kh[:128]) * 0.25
        s_r = _dot_t(qh, kh[128:]) * 0.25
        m = jnp.maximum(jnp.max(s_l, axis=-1, keepdims=True),
                        jnp.max(s_r, axis=-1, keepdims=True))
        e_l = jnp.exp(s_l - m)
        e_r = jnp.exp(s_r - m)
        sm = _chunk_sum(e_l + e_r)
        ov = _dot(e_l, vh[:128]) + _dot(e_r, vh[128:])
        outs.append(ov / sm)
    return jnp.concatenate(outs, axis=1)


def _attn_dec(q, k, v, heads, mask_row):
    outs = []
    for h in range(heads):
        sl = slice(h * _DH, (h + 1) * _DH)
        s = _dot_t(q[:, sl], k[:, sl]) * 0.25
        s = jnp.where(mask_row > 0, s, -1e9)
        m = jnp.max(s, axis=-1, keepdims=True)
        e = jnp.exp(s - m)
        outs.append(_dot(e, v[:, sl]) / _chunk_sum(e))
    return jnp.concatenate(outs, axis=1)


def _rank_row(s_col, s_row, tri, ones_row):
    gt = s_col > s_row
    eq = s_col == s_row
    d = jnp.where(gt, 1.0, jnp.where(eq, tri, 0.0))
    return jax.lax.dot_general(ones_row, d, (((1,), (0,)), ((), ())),
                               preferred_element_type=jnp.float32,
                               precision=_PREC)


def _select_rows(rank_row, feats, k, n):
    m_iota = jax.lax.broadcasted_iota(jnp.int32, (k, n), 0)
    rank_i = jnp.broadcast_to(rank_row, (k, n)).astype(jnp.int32)
    sel = jnp.where(rank_i == m_iota, 1.0, 0.0)
    return _dot_x(sel, feats)


def _body(ped_ref, neis_ref, modes_ref, maskrow_ref, i256_ref, i100_ref,
          t256_ref, t100_ref, *refs):
    out_pred_ref, out_sn_ref = refs[-2], refs[-1]
    p = dict(zip(_PARAM_ORDER, refs[:-2]))
    i256 = i256_ref[...]
    i100 = i100_ref[...]
    t256 = t256_ref[...]
    t100 = t100_ref[...]
    ones256 = jnp.ones((1, _K), jnp.float32)
    ones100 = jnp.ones((1, _PK), jnp.float32)
    for i in range(_GRP):
        _one_batch(i, ped_ref, neis_ref, modes_ref, maskrow_ref, p,
                   i256, i100, t256, t100, ones256, ones100,
                   out_pred_ref, out_sn_ref)


def _one_batch(i, ped_ref, neis_ref, modes_ref, maskrow_ref, p,
               i256, i100, t256, t100, ones256, ones100,
               out_pred_ref, out_sn_ref):
    inp = jnp.concatenate(
        [jnp.broadcast_to(ped_ref[i], (_K, _DIN)), modes_ref[...]], axis=1)
    x = _dot(inp, p['W_emb'][...]) + p['b_emb'][...]

    q = _dot(x, p['enc_Wq'][...]) + p['enc_bq'][...]
    k = _dot(x, p['enc_Wk'][...]) + p['enc_bk'][...]
    v = _dot(x, p['enc_Wv'][...]) + p['enc_bv'][...]
    a = _dot(_attn_enc(q, k, v, _H), p['enc_Wo'][...]) + p['enc_bo'][...]
    x = _ln(x + a, p['enc_ln1_g'][...], p['enc_ln1_b'][...])
    h = jnp.maximum(_dot(x, p['enc_W1'][...]) + p['enc_b1'][...], 0.0)
    h = _dot(h, p['enc_W2'][...]) + p['enc_b2'][...]
    pf = _ln(x + h, p['enc_ln2_g'][...], p['enc_ln2_b'][...])

    sc_col = _dot(pf, p['W_cls'][...]) + p['b_cls'][...]
    sc_row = _col_to_row(sc_col, i256)
    rank = _rank_row(sc_col, sc_row, t256, ones256)
    topf = _select_rows(rank, pf, _PK, _K)

    ne = _dot(neis_ref[i], p['W_nei'][...]) + p['b_nei'][...]
    q2 = _dot(topf, p['dec_Wq'][...]) + p['dec_bq'][...]
    k2 = _dot(ne, p['dec_Wk'][...]) + p['dec_bk'][...]
    v2 = _dot(ne, p['dec_Wv'][...]) + p['dec_bv'][...]
    a2 = _attn_dec(q2, k2, v2, _H, maskrow_ref[i])
    a2 = _dot(a2, p['dec_Wo'][...]) + p['dec_bo'][...]
    x2 = _ln(topf + a2, p['dec_ln1_g'][...], p['dec_ln1_b'][...])
    h2 = jnp.maximum(_dot(x2, p['dec_W1'][...]) + p['dec_b1'][...], 0.0)
    h2 = _dot(h2, p['dec_W2'][...]) + p['dec_b2'][...]
    intf = _ln(x2 + h2, p['dec_ln2_g'][...], p['dec_ln2_b'][...])

    lg = _dot(intf, p['W_cls2'][...]) + p['b_cls2'][...]
    lg_row = _col_to_row(lg, i100)
    mx = jnp.max(lg_row, axis=-1, keepdims=True)
    e = jnp.exp(lg_row - mx)
    sn_row = e / _chunk_sum(e)
    sn_col = _row_to_col(sn_row, i100)
    out_sn_ref[i] = sn_row

    rank2 = _rank_row(sn_col, sn_row, t100, ones100)
    top2 = _select_rows(rank2, intf, _NK, _PK)
    out_pred_ref[i] = _dot(top2, p['W_reg'][...]) + p['b_reg'][...]


def kernel(ped_obs, neis_obs, motion_modes, mask, closest_mode_indices,
           num_k, ped_num_k, params):
    bb = ped_obs.shape[0]
    ped = ped_obs.reshape(bb, 1, _DIN).astype(jnp.float32)
    neis = neis_obs.reshape(bb, _NN, _DIN).astype(jnp.float32)
    modes = motion_modes.reshape(_K, _DMODE).astype(jnp.float32)
    maskrow = mask[:, 0:1, :].astype(jnp.float32)

    ident256 = jnp.eye(_K, dtype=jnp.float32)
    ident100 = jnp.eye(_PK, dtype=jnp.float32)
    tri256 = jnp.triu(jnp.ones((_K, _K), jnp.float32), 1)
    tri100 = jnp.triu(jnp.ones((_PK, _PK), jnp.float32), 1)

    pargs = []
    for name in _PARAM_ORDER:
        w = params[name]
        if w.ndim == 1:
            w = w.reshape(1, -1)
        pargs.append(w.astype(jnp.float32))

    in_specs = [
        pl.BlockSpec((_GRP, 1, _DIN), lambda b: (b, 0, 0)),
        pl.BlockSpec((_GRP, _NN, _DIN), lambda b: (b, 0, 0)),
        pl.BlockSpec((_K, _DMODE), lambda b: (0, 0)),
        pl.BlockSpec((_GRP, 1, _NN), lambda b: (b, 0, 0)),
        pl.BlockSpec((_K, _K), lambda b: (0, 0)),
        pl.BlockSpec((_PK, _PK), lambda b: (0, 0)),
        pl.BlockSpec((_K, _K), lambda b: (0, 0)),
        pl.BlockSpec((_PK, _PK), lambda b: (0, 0)),
    ] + [pl.BlockSpec(w.shape, lambda b, nd=w.ndim: (0,) * nd) for w in pargs]

    out_specs = (
        pl.BlockSpec((_GRP, _NK, 2 * _PRED), lambda b: (b, 0, 0)),
        pl.BlockSpec((_GRP, 1, _PK), lambda b: (b, 0, 0)),
    )
    out_shape = (
        jax.ShapeDtypeStruct((bb, _NK, 2 * _PRED), jnp.float32),
        jax.ShapeDtypeStruct((bb, 1, _PK), jnp.float32),
    )

    pred, sn = pl.pallas_call(
        _body,
        grid=(bb // _GRP,),
        in_specs=in_specs,
        out_specs=out_specs,
        out_shape=out_shape,
        compiler_params=pltpu.CompilerParams(
            dimension_semantics=("parallel",)),
    )(ped, neis, modes, maskrow, ident256, ident100, tri256, tri100, *pargs)
    return (pred, sn.reshape(bb, _PK))


# speedup vs baseline: 2.2649x; 2.2649x over previous
import jax
import jax.numpy as jnp
from jax.experimental import pallas as pl
from jax.experimental.pallas import tpu as pltpu

_B = 64; _K = 256; _NN = 64; _OBS = 8; _PRED = 12; _INS = 2
_E = 64; _H = 4; _FF = 128
_DH = _E // _H
_PK = 100
_NK = 20
_DIN = _OBS * _INS
_DMODE = _PRED * 2
_GRP = 4

_PARAM_ORDER = (
    'W_emb', 'b_emb',
    'enc_Wq', 'enc_bq', 'enc_Wk', 'enc_bk', 'enc_Wv', 'enc_bv',
    'enc_Wo', 'enc_bo', 'enc_ln1_g', 'enc_ln1_b',
    'enc_W1', 'enc_b1', 'enc_W2', 'enc_b2', 'enc_ln2_g', 'enc_ln2_b',
    'dec_Wq', 'dec_bq', 'dec_Wk', 'dec_bk', 'dec_Wv', 'dec_bv',
    'dec_Wo', 'dec_bo', 'dec_ln1_g', 'dec_ln1_b',
    'dec_W1', 'dec_b1', 'dec_W2', 'dec_b2', 'dec_ln2_g', 'dec_ln2_b',
    'W_cls', 'b_cls', 'W_cls2', 'b_cls2', 'W_nei', 'b_nei',
    'W_reg', 'b_reg',
)

_PREC = jax.lax.Precision.HIGHEST


def _dot_f(a, b):
    return jax.lax.dot_general(a, b, (((0,), (0,)), ((), ())),
                               preferred_element_type=jnp.float32)


def _dot_k(a, b):
    return jax.lax.dot_general(a, b, (((1,), (0,)), ((), ())),
                               preferred_element_type=jnp.float32)


def _row_to_col(v, ident):
    return jax.lax.dot_general(ident, v, (((1,), (1,)), ((), ())),
                               preferred_element_type=jnp.float32,
                               precision=_PREC)


def _sub_sum(x):
    r = x.shape[0]
    acc = x[0:8]
    for i in range(1, r // 8):
        acc = acc + x[8 * i:8 * i + 8]
    acc = acc[0:4] + acc[4:8]
    acc = acc[0:2] + acc[2:4]
    return acc[0:1] + acc[1:2]


def _lane_sum(x):
    w = x.shape[-1]
    if w % 8:
        pad = 8 - w % 8
        x = jnp.concatenate(
            [x, jnp.zeros((x.shape[0], pad), x.dtype)], axis=1)
        w += pad
    acc = x[:, 0:8]
    for i in range(1, w // 8):
        acc = acc + x[:, 8 * i:8 * i + 8]
    acc = acc[:, 0:4] + acc[:, 4:8]
    acc = acc[:, 0:2] + acc[:, 2:4]
    return acc[:, 0:1] + acc[:, 1:2]


def _ln(x, g, b):
    n = float(x.shape[0])
    m = _sub_sum(x) / n
    d = x - m
    v = _sub_sum(d * d) / n
    return d / jnp.sqrt(v + 1e-5) * g + b


def _attn_enc(q, k, v):
    outs = []
    for h in range(_H):
        sl = slice(h * _DH, (h + 1) * _DH)
        qh, kh, vh = q[sl], k[sl], v[sl]
        st = _dot_f(kh, qh) * 0.25
        m = jnp.max(st, axis=0, keepdims=True)
        e = jnp.exp(st - m)
        e_l = e[0:128]
        e_r = e[128:256]
        sm = _sub_sum(e_l + e_r)
        ov = _dot_k(vh[:, 0:128], e_l) + _dot_k(vh[:, 128:256], e_r)
        outs.append(ov / sm)
    return jnp.concatenate(outs, axis=0)


def _attn_dec(q, k, v, mask_col):
    outs = []
    for h in range(_H):
        sl = slice(h * _DH, (h + 1) * _DH)
        st = _dot_f(k[sl], q[sl]) * 0.25
        st = jnp.where(mask_col > 0, st, -1e9)
        m = jnp.max(st, axis=0, keepdims=True)
        e = jnp.exp(st - m)
        outs.append(_dot_k(v[sl], e) / _sub_sum(e))
    return jnp.concatenate(outs, axis=0)


def _rank_row(s_col, s_row, tri, ones_row):
    gt = s_col > s_row
    eq = s_col == s_row
    d = jnp.where(gt, 1.0, jnp.where(eq, tri, 0.0))
    return jax.lax.dot_general(ones_row, d, (((1,), (0,)), ((), ())),
                               preferred_element_type=jnp.float32,
                               precision=_PREC)


def _select_cols(rank_row, feats_t, kk, n):
    m_iota = jax.lax.broadcasted_iota(jnp.int32, (kk, n), 0)
    rank_i = jnp.broadcast_to(rank_row, (kk, n)).astype(jnp.int32)
    sel = jnp.where(rank_i == m_iota, 1.0, 0.0)
    return jax.lax.dot_general(feats_t, sel, (((1,), (1,)), ((), ())),
                               preferred_element_type=jnp.float32,
                               precision=_PREC)


def _body(ped_ref, neis_ref, modes_ref, mask_ref, i256_ref, i100_ref,
          t256_ref, t100_ref, *refs):
    out_pred_ref, out_sn_ref = refs[-2], refs[-1]
    p = {name: r[...] for name, r in zip(_PARAM_ORDER, refs[:-2])}
    i256 = i256_ref[...]
    i100 = i100_ref[...]
    t256 = t256_ref[...]
    t100 = t100_ref[...]
    ones256 = jnp.ones((1, _K), jnp.float32)
    ones100 = jnp.ones((1, _PK), jnp.float32)
    for i in range(_GRP):
        _one_batch(i, ped_ref, neis_ref, modes_ref, mask_ref, p,
                   i256, i100, t256, t100, ones256, ones100,
                   out_pred_ref, out_sn_ref)


def _one_batch(i, ped_ref, neis_ref, modes_ref, mask_ref, p,
               i256, i100, t256, t100, ones256, ones100,
               out_pred_ref, out_sn_ref):
    inp = jnp.concatenate(
        [jnp.broadcast_to(ped_ref[i], (_DIN, _K)), modes_ref[...]], axis=0)
    x = _dot_f(p['W_emb'], inp) + p['b_emb']

    q = _dot_f(p['enc_Wq'], x) + p['enc_bq']
    k = _dot_f(p['enc_Wk'], x) + p['enc_bk']
    v = _dot_f(p['enc_Wv'], x) + p['enc_bv']
    a = _dot_f(p['enc_Wo'], _attn_enc(q, k, v)) + p['enc_bo']
    x = _ln(x + a, p['enc_ln1_g'], p['enc_ln1_b'])
    h = jnp.maximum(_dot_f(p['enc_W1'], x) + p['enc_b1'], 0.0)
    h = _dot_f(p['enc_W2'], h) + p['enc_b2']
    pf = _ln(x + h, p['enc_ln2_g'], p['enc_ln2_b'])

    sc_row = _dot_f(p['W_cls'], pf) + p['b_cls']
    sc_col = _row_to_col(sc_row, i256)
    rank = _rank_row(sc_col, sc_row, t256, ones256)
    topf = _select_cols(rank, pf, _PK, _K)

    ne = _dot_f(p['W_nei'], neis_ref[i]) + p['b_nei']
    q2 = _dot_f(p['dec_Wq'], topf) + p['dec_bq']
    k2 = _dot_f(p['dec_Wk'], ne) + p['dec_bk']
    v2 = _dot_f(p['dec_Wv'], ne) + p['dec_bv']
    a2 = _dot_f(p['dec_Wo'], _attn_dec(q2, k2, v2, mask_ref[i])) + p['dec_bo']
    x2 = _ln(topf + a2, p['dec_ln1_g'], p['dec_ln1_b'])
    h2 = jnp.maximum(_dot_f(p['dec_W1'], x2) + p['dec_b1'], 0.0)
    h2 = _dot_f(p['dec_W2'], h2) + p['dec_b2']
    intf = _ln(x2 + h2, p['dec_ln2_g'], p['dec_ln2_b'])

    lg_row = _dot_f(p['W_cls2'], intf) + p['b_cls2']
    mx = jnp.max(lg_row, axis=-1, keepdims=True)
    e = jnp.exp(lg_row - mx)
    sn_row = e / _lane_sum(e)
    sn_col = _row_to_col(sn_row, i100)
    out_sn_ref[i] = sn_row

    rank2 = _rank_row(sn_col, sn_row, t100, ones100)
    top2 = _select_cols(rank2, intf, _NK, _PK)
    out_pred_ref[i] = _dot_f(p['W_reg'], top2) + p['b_reg']


def kernel(ped_obs, neis_obs, motion_modes, mask, closest_mode_indices,
           num_k, ped_num_k, params):
    bb = ped_obs.shape[0]
    f32 = jnp.float32
    ped_t = ped_obs.reshape(bb, 1, _DIN).swapaxes(1, 2).astype(f32)
    neis_t = neis_obs.reshape(bb, _NN, _DIN).swapaxes(1, 2).astype(f32)
    modes_t = motion_modes.reshape(_K, _DMODE).T.astype(f32)
    mask_col = mask[:, 0, :, None].astype(f32)

    ident256 = jnp.eye(_K, dtype=f32)
    ident100 = jnp.eye(_PK, dtype=f32)
    tri256 = jnp.triu(jnp.ones((_K, _K), f32), 1)
    tri100 = jnp.triu(jnp.ones((_PK, _PK), f32), 1)

    pargs = []
    for name in _PARAM_ORDER:
        w = params[name].astype(f32)
        if w.ndim == 1:
            w = w.reshape(-1, 1)
        pargs.append(w)

    in_specs = [
        pl.BlockSpec((_GRP, _DIN, 1), lambda b: (b, 0, 0)),
        pl.BlockSpec((_GRP, _DIN, _NN), lambda b: (b, 0, 0)),
        pl.BlockSpec((_DMODE, _K), lambda b: (0, 0)),
        pl.BlockSpec((_GRP, _NN, 1), lambda b: (b, 0, 0)),
        pl.BlockSpec((_K, _K), lambda b: (0, 0)),
        pl.BlockSpec((_PK, _PK), lambda b: (0, 0)),
        pl.BlockSpec((_K, _K), lambda b: (0, 0)),
        pl.BlockSpec((_PK, _PK), lambda b: (0, 0)),
    ] + [pl.BlockSpec(w.shape, lambda b, nd=w.ndim: (0,) * nd) for w in pargs]

    out_specs = (
        pl.BlockSpec((_GRP, 2 * _PRED, _NK), lambda b: (b, 0, 0)),
        pl.BlockSpec((_GRP, 1, _PK), lambda b: (b, 0, 0)),
    )
    out_shape = (
        jax.ShapeDtypeStruct((bb, 2 * _PRED, _NK), f32),
        jax.ShapeDtypeStruct((bb, 1, _PK), f32),
    )

    pred_t, sn = pl.pallas_call(
        _body,
        grid=(bb // _GRP,),
        in_specs=in_specs,
        out_specs=out_specs,
        out_shape=out_shape,
        compiler_params=pltpu.CompilerParams(
            dimension_semantics=("parallel",)),
    )(ped_t, neis_t, modes_t, mask_col, ident256, ident100, tri256, tri100,
      *pargs)
    return (pred_t.swapaxes(1, 2), sn.reshape(bb, _PK))


# speedup vs baseline: 3.0742x; 1.3573x over previous
import jax
import jax.numpy as jnp
from jax.experimental import pallas as pl
from jax.experimental.pallas import tpu as pltpu

_B = 64; _K = 256; _NN = 64; _OBS = 8; _PRED = 12; _INS = 2
_E = 64; _H = 4; _FF = 128
_DH = _E // _H
_PK = 100
_NK = 20
_DIN = _OBS * _INS
_DMODE = _PRED * 2
_GRP = 4

_PARAM_ORDER = (
    'W_emb', 'b_emb',
    'enc_Wq', 'enc_bq', 'enc_Wk', 'enc_bk', 'enc_Wv', 'enc_bv',
    'enc_Wo', 'enc_bo', 'enc_ln1_g', 'enc_ln1_b',
    'enc_W1', 'enc_b1', 'enc_W2', 'enc_b2', 'enc_ln2_g', 'enc_ln2_b',
    'dec_Wq', 'dec_bq', 'dec_Wk', 'dec_bk', 'dec_Wv', 'dec_bv',
    'dec_Wo', 'dec_bo', 'dec_ln1_g', 'dec_ln1_b',
    'dec_W1', 'dec_b1', 'dec_W2', 'dec_b2', 'dec_ln2_g', 'dec_ln2_b',
    'W_cls', 'b_cls', 'W_cls2', 'b_cls2', 'W_nei', 'b_nei',
    'W_reg', 'b_reg',
)

_PREC = jax.lax.Precision.HIGHEST


def _dot_f(a, b):
    return jax.lax.dot_general(a, b, (((0,), (0,)), ((), ())),
                               preferred_element_type=jnp.float32)


def _dot_k(a, b):
    return jax.lax.dot_general(a, b, (((1,), (0,)), ((), ())),
                               preferred_element_type=jnp.float32)


def _row_to_col(v, ident):
    return jax.lax.dot_general(ident, v, (((1,), (1,)), ((), ())),
                               preferred_element_type=jnp.float32,
                               precision=_PREC)


def _sub_sum(x):
    r = x.shape[0]
    acc = x[0:8]
    for i in range(1, r // 8):
        acc = acc + x[8 * i:8 * i + 8]
    acc = acc[0:4] + acc[4:8]
    acc = acc[0:2] + acc[2:4]
    return acc[0:1] + acc[1:2]


def _lane_sum(x):
    w = x.shape[-1]
    if w % 8:
        pad = 8 - w % 8
        x = jnp.concatenate(
            [x, jnp.zeros((x.shape[0], pad), x.dtype)], axis=1)
        w += pad
    acc = x[:, 0:8]
    for i in range(1, w // 8):
        acc = acc + x[:, 8 * i:8 * i + 8]
    acc = acc[:, 0:4] + acc[:, 4:8]
    acc = acc[:, 0:2] + acc[:, 2:4]
    return acc[:, 0:1] + acc[:, 1:2]


def _ln(x, g, b):
    n = float(x.shape[0])
    m = _sub_sum(x) / n
    d = x - m
    v = _sub_sum(d * d) / n
    return d / jnp.sqrt(v + 1e-5) * g + b


def _attn_enc(q, k, v):
    outs = []
    for h in range(_H):
        sl = slice(h * _DH, (h + 1) * _DH)
        qh, kh, vh = q[sl], k[sl], v[sl]
        st = jnp.concatenate(
            [_dot_f(kh[:, b * _K:(b + 1) * _K], qh[:, b * _K:(b + 1) * _K])
             for b in range(_GRP)], axis=1) * 0.25
        m = jnp.max(st, axis=0, keepdims=True)
        e = jnp.exp(st - m)
        sm = _sub_sum(e[0:128] + e[128:256])
        ovs = []
        for b in range(_GRP):
            csl = slice(b * _K, (b + 1) * _K)
            vb = vh[:, csl]
            ovs.append(_dot_k(vb[:, 0:128], e[0:128, csl])
                       + _dot_k(vb[:, 128:256], e[128:256, csl]))
        outs.append(jnp.concatenate(ovs, axis=1) / sm)
    return jnp.concatenate(outs, axis=0)


def _attn_dec(q, k, v, mask_col):
    outs = []
    for h in range(_H):
        sl = slice(h * _DH, (h + 1) * _DH)
        st = _dot_f(k[sl], q[sl]) * 0.25
        st = jnp.where(mask_col > 0, st, -1e9)
        m = jnp.max(st, axis=0, keepdims=True)
        e = jnp.exp(st - m)
        outs.append(_dot_k(v[sl], e) / _sub_sum(e))
    return jnp.concatenate(outs, axis=0)


def _rank_row(s_col, s_row, tri, ones_row):
    gt = s_col > s_row
    eq = s_col == s_row
    d = jnp.where(gt, 1.0, jnp.where(eq, tri, 0.0))
    return jax.lax.dot_general(ones_row, d, (((1,), (0,)), ((), ())),
                               preferred_element_type=jnp.float32,
                               precision=_PREC)


def _select_cols(rank_row, feats_t, kk, n):
    m_iota = jax.lax.broadcasted_iota(jnp.int32, (kk, n), 0)
    rank_i = jnp.broadcast_to(rank_row, (kk, n)).astype(jnp.int32)
    sel = jnp.where(rank_i == m_iota, 1.0, 0.0)
    return jax.lax.dot_general(feats_t, sel, (((1,), (1,)), ((), ())),
                               preferred_element_type=jnp.float32,
                               precision=_PREC)


def _body(ped_ref, neis_ref, modes_ref, mask_ref, i256_ref, i100_ref,
          t256_ref, t100_ref, *refs):
    out_pred_ref, out_sn_ref = refs[-2], refs[-1]
    p = {name: r[...] for name, r in zip(_PARAM_ORDER, refs[:-2])}
    i256 = i256_ref[...]
    i100 = i100_ref[...]
    t256 = t256_ref[...]
    t100 = t100_ref[...]
    ones256 = jnp.ones((1, _K), jnp.float32)
    ones100 = jnp.ones((1, _PK), jnp.float32)

    modes = modes_ref[...]
    inp = jnp.concatenate(
        [jnp.concatenate(
            [jnp.broadcast_to(ped_ref[i], (_DIN, _K)) for i in range(_GRP)],
            axis=1)]
        + [jnp.concatenate([modes] * _GRP, axis=1)], axis=0)
    x = _dot_f(p['W_emb'], inp) + p['b_emb']
    q = _dot_f(p['enc_Wq'], x) + p['enc_bq']
    k = _dot_f(p['enc_Wk'], x) + p['enc_bk']
    v = _dot_f(p['enc_Wv'], x) + p['enc_bv']
    a = _dot_f(p['enc_Wo'], _attn_enc(q, k, v)) + p['enc_bo']
    x = _ln(x + a, p['enc_ln1_g'], p['enc_ln1_b'])
    h = jnp.maximum(_dot_f(p['enc_W1'], x) + p['enc_b1'], 0.0)
    h = _dot_f(p['enc_W2'], h) + p['enc_b2']
    pf = _ln(x + h, p['enc_ln2_g'], p['enc_ln2_b'])
    sc_row_cat = _dot_f(p['W_cls'], pf) + p['b_cls']

    for i in range(_GRP):
        csl = slice(i * _K, (i + 1) * _K)
        _one_batch(i, pf[:, csl], sc_row_cat[:, csl], neis_ref, mask_ref, p,
                   i256, i100, t256, t100, ones256, ones100,
                   out_pred_ref, out_sn_ref)


def _one_batch(i, pf, sc_row, neis_ref, mask_ref, p,
               i256, i100, t256, t100, ones256, ones100,
               out_pred_ref, out_sn_ref):
    sc_col = _row_to_col(sc_row, i256)
    rank = _rank_row(sc_col, sc_row, t256, ones256)
    topf = _select_cols(rank, pf, _PK, _K)

    ne = _dot_f(p['W_nei'], neis_ref[i]) + p['b_nei']
    q2 = _dot_f(p['dec_Wq'], topf) + p['dec_bq']
    k2 = _dot_f(p['dec_Wk'], ne) + p['dec_bk']
    v2 = _dot_f(p['dec_Wv'], ne) + p['dec_bv']
    a2 = _dot_f(p['dec_Wo'], _attn_dec(q2, k2, v2, mask_ref[i])) + p['dec_bo']
    x2 = _ln(topf + a2, p['dec_ln1_g'], p['dec_ln1_b'])
    h2 = jnp.maximum(_dot_f(p['dec_W1'], x2) + p['dec_b1'], 0.0)
    h2 = _dot_f(p['dec_W2'], h2) + p['dec_b2']
    intf = _ln(x2 + h2, p['dec_ln2_g'], p['dec_ln2_b'])

    lg_row = _dot_f(p['W_cls2'], intf) + p['b_cls2']
    mx = jnp.max(lg_row, axis=-1, keepdims=True)
    e = jnp.exp(lg_row - mx)
    sn_row = e / _lane_sum(e)
    sn_col = _row_to_col(sn_row, i100)
    out_sn_ref[i] = sn_row

    rank2 = _rank_row(sn_col, sn_row, t100, ones100)
    top2 = _select_cols(rank2, intf, _NK, _PK)
    out_pred_ref[i] = _dot_f(p['W_reg'], top2) + p['b_reg']


def kernel(ped_obs, neis_obs, motion_modes, mask, closest_mode_indices,
           num_k, ped_num_k, params):
    bb = ped_obs.shape[0]
    f32 = jnp.float32
    ped_t = ped_obs.reshape(bb, 1, _DIN).swapaxes(1, 2).astype(f32)
    neis_t = neis_obs.reshape(bb, _NN, _DIN).swapaxes(1, 2).astype(f32)
    modes_t = motion_modes.reshape(_K, _DMODE).T.astype(f32)
    mask_col = mask[:, 0, :, None].astype(f32)

    ident256 = jnp.eye(_K, dtype=f32)
    ident100 = jnp.eye(_PK, dtype=f32)
    tri256 = jnp.triu(jnp.ones((_K, _K), f32), 1)
    tri100 = jnp.triu(jnp.ones((_PK, _PK), f32), 1)

    pargs = []
    for name in _PARAM_ORDER:
        w = params[name].astype(f32)
        if w.ndim == 1:
            w = w.reshape(-1, 1)
        pargs.append(w)

    in_specs = [
        pl.BlockSpec((_GRP, _DIN, 1), lambda b: (b, 0, 0)),
        pl.BlockSpec((_GRP, _DIN, _NN), lambda b: (b, 0, 0)),
        pl.BlockSpec((_DMODE, _K), lambda b: (0, 0)),
        pl.BlockSpec((_GRP, _NN, 1), lambda b: (b, 0, 0)),
        pl.BlockSpec((_K, _K), lambda b: (0, 0)),
        pl.BlockSpec((_PK, _PK), lambda b: (0, 0)),
        pl.BlockSpec((_K, _K), lambda b: (0, 0)),
        pl.BlockSpec((_PK, _PK), lambda b: (0, 0)),
    ] + [pl.BlockSpec(w.shape, lambda b, nd=w.ndim: (0,) * nd) for w in pargs]

    out_specs = (
        pl.BlockSpec((_GRP, 2 * _PRED, _NK), lambda b: (b, 0, 0)),
        pl.BlockSpec((_GRP, 1, _PK), lambda b: (b, 0, 0)),
    )
    out_shape = (
        jax.ShapeDtypeStruct((bb, 2 * _PRED, _NK), f32),
        jax.ShapeDtypeStruct((bb, 1, _PK), f32),
    )

    pred_t, sn = pl.pallas_call(
        _body,
        grid=(bb // _GRP,),
        in_specs=in_specs,
        out_specs=out_specs,
        out_shape=out_shape,
        compiler_params=pltpu.CompilerParams(
            dimension_semantics=("parallel",)),
    )(ped_t, neis_t, modes_t, mask_col, ident256, ident100, tri256, tri100,
      *pargs)
    return (pred_t.swapaxes(1, 2), sn.reshape(bb, _PK))


# speedup vs baseline: 3.2098x; 1.0441x over previous
import jax
import jax.numpy as jnp
from jax.experimental import pallas as pl
from jax.experimental.pallas import tpu as pltpu

_B = 64; _K = 256; _NN = 64; _OBS = 8; _PRED = 12; _INS = 2
_E = 64; _H = 4; _FF = 128
_DH = _E // _H
_PK = 100
_NK = 20
_DIN = _OBS * _INS
_DMODE = _PRED * 2
_GRP = 8

_PARAM_ORDER = (
    'W_emb', 'b_emb',
    'enc_Wq', 'enc_bq', 'enc_Wk', 'enc_bk', 'enc_Wv', 'enc_bv',
    'enc_Wo', 'enc_bo', 'enc_ln1_g', 'enc_ln1_b',
    'enc_W1', 'enc_b1', 'enc_W2', 'enc_b2', 'enc_ln2_g', 'enc_ln2_b',
    'dec_Wq', 'dec_bq', 'dec_Wk', 'dec_bk', 'dec_Wv', 'dec_bv',
    'dec_Wo', 'dec_bo', 'dec_ln1_g', 'dec_ln1_b',
    'dec_W1', 'dec_b1', 'dec_W2', 'dec_b2', 'dec_ln2_g', 'dec_ln2_b',
    'W_cls', 'b_cls', 'W_cls2', 'b_cls2', 'W_nei', 'b_nei',
    'W_reg', 'b_reg',
)

_PREC = jax.lax.Precision.HIGHEST


def _dot_f(a, b):
    return jax.lax.dot_general(a, b, (((0,), (0,)), ((), ())),
                               preferred_element_type=jnp.float32)


def _dot_k(a, b):
    return jax.lax.dot_general(a, b, (((1,), (0,)), ((), ())),
                               preferred_element_type=jnp.float32)


def _row_to_col(v, ident):
    return jax.lax.dot_general(ident, v, (((1,), (1,)), ((), ())),
                               preferred_element_type=jnp.float32,
                               precision=_PREC)


def _sub_sum(x):
    r = x.shape[0]
    acc = x[0:8]
    for i in range(1, r // 8):
        acc = acc + x[8 * i:8 * i + 8]
    acc = acc[0:4] + acc[4:8]
    acc = acc[0:2] + acc[2:4]
    return acc[0:1] + acc[1:2]


def _lane_sum(x):
    w = x.shape[-1]
    if w % 8:
        pad = 8 - w % 8
        x = jnp.concatenate(
            [x, jnp.zeros((x.shape[0], pad), x.dtype)], axis=1)
        w += pad
    acc = x[:, 0:8]
    for i in range(1, w // 8):
        acc = acc + x[:, 8 * i:8 * i + 8]
    acc = acc[:, 0:4] + acc[:, 4:8]
    acc = acc[:, 0:2] + acc[:, 2:4]
    return acc[:, 0:1] + acc[:, 1:2]


def _ln(x, g, b):
    n = float(x.shape[0])
    m = _sub_sum(x) / n
    d = x - m
    v = _sub_sum(d * d) / n
    return d / jnp.sqrt(v + 1e-5) * g + b


def _attn_enc(q, k, v):
    outs = []
    for h in range(_H):
        sl = slice(h * _DH, (h + 1) * _DH)
        qh, kh, vh = q[sl], k[sl], v[sl]
        st = jnp.concatenate(
            [_dot_f(kh[:, b * _K:(b + 1) * _K], qh[:, b * _K:(b + 1) * _K])
             for b in range(_GRP)], axis=1) * 0.25
        m = jnp.max(st, axis=0, keepdims=True)
        e = jnp.exp(st - m)
        sm = _sub_sum(e[0:128] + e[128:256])
        ovs = []
        for b in range(_GRP):
            csl = slice(b * _K, (b + 1) * _K)
            vb = vh[:, csl]
            ovs.append(_dot_k(vb[:, 0:128], e[0:128, csl])
                       + _dot_k(vb[:, 128:256], e[128:256, csl]))
        outs.append(jnp.concatenate(ovs, axis=1) / sm)
    return jnp.concatenate(outs, axis=0)


def _attn_dec(q, k, v, mask_col):
    outs = []
    for h in range(_H):
        sl = slice(h * _DH, (h + 1) * _DH)
        st = _dot_f(k[sl], q[sl]) * 0.25
        st = jnp.where(mask_col > 0, st, -1e9)
        m = jnp.max(st, axis=0, keepdims=True)
        e = jnp.exp(st - m)
        outs.append(_dot_k(v[sl], e) / _sub_sum(e))
    return jnp.concatenate(outs, axis=0)


def _rank_row(s_col, s_row, tri, ones_row):
    gt = s_col > s_row
    eq = s_col == s_row
    d = jnp.where(gt, 1.0, jnp.where(eq, tri, 0.0))
    return jax.lax.dot_general(ones_row, d, (((1,), (0,)), ((), ())),
                               preferred_element_type=jnp.float32,
                               precision=_PREC)


def _select_cols(rank_row, feats_t, kk, n):
    m_iota = jax.lax.broadcasted_iota(jnp.int32, (kk, n), 0)
    rank_i = jnp.broadcast_to(rank_row, (kk, n)).astype(jnp.int32)
    sel = jnp.where(rank_i == m_iota, 1.0, 0.0)
    return jax.lax.dot_general(feats_t, sel, (((1,), (1,)), ((), ())),
                               preferred_element_type=jnp.float32,
                               precision=_PREC)


def _body(ped_ref, neis_ref, modes_ref, mask_ref, i256_ref, i100_ref,
          t256_ref, t100_ref, *refs):
    out_pred_ref, out_sn_ref = refs[-2], refs[-1]
    p = {name: r[...] for name, r in zip(_PARAM_ORDER, refs[:-2])}
    i256 = i256_ref[...]
    i100 = i100_ref[...]
    t256 = t256_ref[...]
    t100 = t100_ref[...]
    ones256 = jnp.ones((1, _K), jnp.float32)
    ones100 = jnp.ones((1, _PK), jnp.float32)

    modes = modes_ref[...]
    inp = jnp.concatenate(
        [jnp.concatenate(
            [jnp.broadcast_to(ped_ref[i], (_DIN, _K)) for i in range(_GRP)],
            axis=1)]
        + [jnp.concatenate([modes] * _GRP, axis=1)], axis=0)
    x = _dot_f(p['W_emb'], inp) + p['b_emb']
    q = _dot_f(p['enc_Wq'], x) + p['enc_bq']
    k = _dot_f(p['enc_Wk'], x) + p['enc_bk']
    v = _dot_f(p['enc_Wv'], x) + p['enc_bv']
    a = _dot_f(p['enc_Wo'], _attn_enc(q, k, v)) + p['enc_bo']
    x = _ln(x + a, p['enc_ln1_g'], p['enc_ln1_b'])
    h = jnp.maximum(_dot_f(p['enc_W1'], x) + p['enc_b1'], 0.0)
    h = _dot_f(p['enc_W2'], h) + p['enc_b2']
    pf = _ln(x + h, p['enc_ln2_g'], p['enc_ln2_b'])
    sc_row_cat = _dot_f(p['W_cls'], pf) + p['b_cls']

    for i in range(_GRP):
        csl = slice(i * _K, (i + 1) * _K)
        _one_batch(i, pf[:, csl], sc_row_cat[:, csl], neis_ref, mask_ref, p,
                   i256, i100, t256, t100, ones256, ones100,
                   out_pred_ref, out_sn_ref)


def _one_batch(i, pf, sc_row, neis_ref, mask_ref, p,
               i256, i100, t256, t100, ones256, ones100,
               out_pred_ref, out_sn_ref):
    sc_col = _row_to_col(sc_row, i256)
    rank = _rank_row(sc_col, sc_row, t256, ones256)
    topf = _select_cols(rank, pf, _PK, _K)

    ne = _dot_f(p['W_nei'], neis_ref[i]) + p['b_nei']
    q2 = _dot_f(p['dec_Wq'], topf) + p['dec_bq']
    k2 = _dot_f(p['dec_Wk'], ne) + p['dec_bk']
    v2 = _dot_f(p['dec_Wv'], ne) + p['dec_bv']
    a2 = _dot_f(p['dec_Wo'], _attn_dec(q2, k2, v2, mask_ref[i])) + p['dec_bo']
    x2 = _ln(topf + a2, p['dec_ln1_g'], p['dec_ln1_b'])
    h2 = jnp.maximum(_dot_f(p['dec_W1'], x2) + p['dec_b1'], 0.0)
    h2 = _dot_f(p['dec_W2'], h2) + p['dec_b2']
    intf = _ln(x2 + h2, p['dec_ln2_g'], p['dec_ln2_b'])

    lg_row = _dot_f(p['W_cls2'], intf) + p['b_cls2']
    mx = jnp.max(lg_row, axis=-1, keepdims=True)
    e = jnp.exp(lg_row - mx)
    sn_row = e / _lane_sum(e)
    sn_col = _row_to_col(sn_row, i100)
    out_sn_ref[i] = sn_row

    rank2 = _rank_row(sn_col, sn_row, t100, ones100)
    top2 = _select_cols(rank2, intf, _NK, _PK)
    out_pred_ref[i] = _dot_f(p['W_reg'], top2) + p['b_reg']


def kernel(ped_obs, neis_obs, motion_modes, mask, closest_mode_indices,
           num_k, ped_num_k, params):
    bb = ped_obs.shape[0]
    f32 = jnp.float32
    ped_t = ped_obs.reshape(bb, 1, _DIN).swapaxes(1, 2).astype(f32)
    neis_t = neis_obs.reshape(bb, _NN, _DIN).swapaxes(1, 2).astype(f32)
    modes_t = motion_modes.reshape(_K, _DMODE).T.astype(f32)
    mask_col = mask[:, 0, :, None].astype(f32)

    ident256 = jnp.eye(_K, dtype=f32)
    ident100 = jnp.eye(_PK, dtype=f32)
    tri256 = jnp.triu(jnp.ones((_K, _K), f32), 1)
    tri100 = jnp.triu(jnp.ones((_PK, _PK), f32), 1)

    pargs = []
    for name in _PARAM_ORDER:
        w = params[name].astype(f32)
        if w.ndim == 1:
            w = w.reshape(-1, 1)
        pargs.append(w)

    in_specs = [
        pl.BlockSpec((_GRP, _DIN, 1), lambda b: (b, 0, 0)),
        pl.BlockSpec((_GRP, _DIN, _NN), lambda b: (b, 0, 0)),
        pl.BlockSpec((_DMODE, _K), lambda b: (0, 0)),
        pl.BlockSpec((_GRP, _NN, 1), lambda b: (b, 0, 0)),
        pl.BlockSpec((_K, _K), lambda b: (0, 0)),
        pl.BlockSpec((_PK, _PK), lambda b: (0, 0)),
        pl.BlockSpec((_K, _K), lambda b: (0, 0)),
        pl.BlockSpec((_PK, _PK), lambda b: (0, 0)),
    ] + [pl.BlockSpec(w.shape, lambda b, nd=w.ndim: (0,) * nd) for w in pargs]

    out_specs = (
        pl.BlockSpec((_GRP, 2 * _PRED, _NK), lambda b: (b, 0, 0)),
        pl.BlockSpec((_GRP, 1, _PK), lambda b: (b, 0, 0)),
    )
    out_shape = (
        jax.ShapeDtypeStruct((bb, 2 * _PRED, _NK), f32),
        jax.ShapeDtypeStruct((bb, 1, _PK), f32),
    )

    pred_t, sn = pl.pallas_call(
        _body,
        grid=(bb // _GRP,),
        in_specs=in_specs,
        out_specs=out_specs,
        out_shape=out_shape,
        compiler_params=pltpu.CompilerParams(
            dimension_semantics=("parallel",)),
    )(ped_t, neis_t, modes_t, mask_col, ident256, ident100, tri256, tri100,
      *pargs)
    return (pred_t.swapaxes(1, 2), sn.reshape(bb, _PK))


# speedup vs baseline: 3.2798x; 1.0218x over previous
import jax
import jax.numpy as jnp
from jax.experimental import pallas as pl
from jax.experimental.pallas import tpu as pltpu

_B = 64; _K = 256; _NN = 64; _OBS = 8; _PRED = 12; _INS = 2
_E = 64; _H = 4; _FF = 128
_DH = _E // _H
_PK = 100
_NK = 20
_DIN = _OBS * _INS
_DMODE = _PRED * 2
_GRP = 16

_PARAM_ORDER = (
    'W_emb', 'b_emb',
    'enc_Wq', 'enc_bq', 'enc_Wk', 'enc_bk', 'enc_Wv', 'enc_bv',
    'enc_Wo', 'enc_bo', 'enc_ln1_g', 'enc_ln1_b',
    'enc_W1', 'enc_b1', 'enc_W2', 'enc_b2', 'enc_ln2_g', 'enc_ln2_b',
    'dec_Wq', 'dec_bq', 'dec_Wk', 'dec_bk', 'dec_Wv', 'dec_bv',
    'dec_Wo', 'dec_bo', 'dec_ln1_g', 'dec_ln1_b',
    'dec_W1', 'dec_b1', 'dec_W2', 'dec_b2', 'dec_ln2_g', 'dec_ln2_b',
    'W_cls', 'b_cls', 'W_cls2', 'b_cls2', 'W_nei', 'b_nei',
    'W_reg', 'b_reg',
)

_PREC = jax.lax.Precision.HIGHEST


def _dot_f(a, b):
    return jax.lax.dot_general(a, b, (((0,), (0,)), ((), ())),
                               preferred_element_type=jnp.float32)


def _dot_k(a, b):
    return jax.lax.dot_general(a, b, (((1,), (0,)), ((), ())),
                               preferred_element_type=jnp.float32)


def _row_to_col(v, ident):
    return jax.lax.dot_general(ident, v, (((1,), (1,)), ((), ())),
                               preferred_element_type=jnp.float32,
                               precision=_PREC)


def _sub_sum(x):
    r = x.shape[0]
    acc = x[0:8]
    for i in range(1, r // 8):
        acc = acc + x[8 * i:8 * i + 8]
    acc = acc[0:4] + acc[4:8]
    acc = acc[0:2] + acc[2:4]
    return acc[0:1] + acc[1:2]


def _lane_sum(x):
    w = x.shape[-1]
    if w % 8:
        pad = 8 - w % 8
        x = jnp.concatenate(
            [x, jnp.zeros((x.shape[0], pad), x.dtype)], axis=1)
        w += pad
    acc = x[:, 0:8]
    for i in range(1, w // 8):
        acc = acc + x[:, 8 * i:8 * i + 8]
    acc = acc[:, 0:4] + acc[:, 4:8]
    acc = acc[:, 0:2] + acc[:, 2:4]
    return acc[:, 0:1] + acc[:, 1:2]


def _ln(x, g, b):
    n = float(x.shape[0])
    m = _sub_sum(x) / n
    d = x - m
    v = _sub_sum(d * d) / n
    return d / jnp.sqrt(v + 1e-5) * g + b


def _attn_enc(q, k, v):
    outs = []
    for h in range(_H):
        sl = slice(h * _DH, (h + 1) * _DH)
        qh, kh, vh = q[sl], k[sl], v[sl]
        st = jnp.concatenate(
            [_dot_f(kh[:, b * _K:(b + 1) * _K], qh[:, b * _K:(b + 1) * _K])
             for b in range(_GRP)], axis=1) * 0.25
        m = jnp.max(st, axis=0, keepdims=True)
        e = jnp.exp(st - m)
        sm = _sub_sum(e[0:128] + e[128:256])
        ovs = []
        for b in range(_GRP):
            csl = slice(b * _K, (b + 1) * _K)
            vb = vh[:, csl]
            ovs.append(_dot_k(vb[:, 0:128], e[0:128, csl])
                       + _dot_k(vb[:, 128:256], e[128:256, csl]))
        outs.append(jnp.concatenate(ovs, axis=1) / sm)
    return jnp.concatenate(outs, axis=0)


def _attn_dec(q, k, v, mask_col):
    outs = []
    for h in range(_H):
        sl = slice(h * _DH, (h + 1) * _DH)
        st = _dot_f(k[sl], q[sl]) * 0.25
        st = jnp.where(mask_col > 0, st, -1e9)
        m = jnp.max(st, axis=0, keepdims=True)
        e = jnp.exp(st - m)
        outs.append(_dot_k(v[sl], e) / _sub_sum(e))
    return jnp.concatenate(outs, axis=0)


def _rank_row(s_col, s_row, tri, ones_row):
    gt = s_col > s_row
    eq = s_col == s_row
    d = jnp.where(gt, 1.0, jnp.where(eq, tri, 0.0))
    return jax.lax.dot_general(ones_row, d, (((1,), (0,)), ((), ())),
                               preferred_element_type=jnp.float32,
                               precision=_PREC)


def _select_cols(rank_row, feats_t, kk, n):
    m_iota = jax.lax.broadcasted_iota(jnp.int32, (kk, n), 0)
    rank_i = jnp.broadcast_to(rank_row, (kk, n)).astype(jnp.int32)
    sel = jnp.where(rank_i == m_iota, 1.0, 0.0)
    return jax.lax.dot_general(feats_t, sel, (((1,), (1,)), ((), ())),
                               preferred_element_type=jnp.float32,
                               precision=_PREC)


def _body(ped_ref, neis_ref, modes_ref, mask_ref, i256_ref, i100_ref,
          t256_ref, t100_ref, *refs):
    out_pred_ref, out_sn_ref = refs[-2], refs[-1]
    p = {name: r[...] for name, r in zip(_PARAM_ORDER, refs[:-2])}
    i256 = i256_ref[...]
    i100 = i100_ref[...]
    t256 = t256_ref[...]
    t100 = t100_ref[...]
    ones256 = jnp.ones((1, _K), jnp.float32)
    ones100 = jnp.ones((1, _PK), jnp.float32)

    modes = modes_ref[...]
    inp = jnp.concatenate(
        [jnp.concatenate(
            [jnp.broadcast_to(ped_ref[i], (_DIN, _K)) for i in range(_GRP)],
            axis=1)]
        + [jnp.concatenate([modes] * _GRP, axis=1)], axis=0)
    x = _dot_f(p['W_emb'], inp) + p['b_emb']
    q = _dot_f(p['enc_Wq'], x) + p['enc_bq']
    k = _dot_f(p['enc_Wk'], x) + p['enc_bk']
    v = _dot_f(p['enc_Wv'], x) + p['enc_bv']
    a = _dot_f(p['enc_Wo'], _attn_enc(q, k, v)) + p['enc_bo']
    x = _ln(x + a, p['enc_ln1_g'], p['enc_ln1_b'])
    h = jnp.maximum(_dot_f(p['enc_W1'], x) + p['enc_b1'], 0.0)
    h = _dot_f(p['enc_W2'], h) + p['enc_b2']
    pf = _ln(x + h, p['enc_ln2_g'], p['enc_ln2_b'])
    sc_row_cat = _dot_f(p['W_cls'], pf) + p['b_cls']

    for i in range(_GRP):
        csl = slice(i * _K, (i + 1) * _K)
        _one_batch(i, pf[:, csl], sc_row_cat[:, csl], neis_ref, mask_ref, p,
                   i256, i100, t256, t100, ones256, ones100,
                   out_pred_ref, out_sn_ref)


def _one_batch(i, pf, sc_row, neis_ref, mask_ref, p,
               i256, i100, t256, t100, ones256, ones100,
               out_pred_ref, out_sn_ref):
    sc_col = _row_to_col(sc_row, i256)
    rank = _rank_row(sc_col, sc_row, t256, ones256)
    topf = _select_cols(rank, pf, _PK, _K)

    ne = _dot_f(p['W_nei'], neis_ref[i]) + p['b_nei']
    q2 = _dot_f(p['dec_Wq'], topf) + p['dec_bq']
    k2 = _dot_f(p['dec_Wk'], ne) + p['dec_bk']
    v2 = _dot_f(p['dec_Wv'], ne) + p['dec_bv']
    a2 = _dot_f(p['dec_Wo'], _attn_dec(q2, k2, v2, mask_ref[i])) + p['dec_bo']
    x2 = _ln(topf + a2, p['dec_ln1_g'], p['dec_ln1_b'])
    h2 = jnp.maximum(_dot_f(p['dec_W1'], x2) + p['dec_b1'], 0.0)
    h2 = _dot_f(p['dec_W2'], h2) + p['dec_b2']
    intf = _ln(x2 + h2, p['dec_ln2_g'], p['dec_ln2_b'])

    lg_row = _dot_f(p['W_cls2'], intf) + p['b_cls2']
    mx = jnp.max(lg_row, axis=-1, keepdims=True)
    e = jnp.exp(lg_row - mx)
    sn_row = e / _lane_sum(e)
    sn_col = _row_to_col(sn_row, i100)
    out_sn_ref[i] = sn_row

    rank2 = _rank_row(sn_col, sn_row, t100, ones100)
    top2 = _select_cols(rank2, intf, _NK, _PK)
    out_pred_ref[i] = _dot_f(p['W_reg'], top2) + p['b_reg']


def kernel(ped_obs, neis_obs, motion_modes, mask, closest_mode_indices,
           num_k, ped_num_k, params):
    bb = ped_obs.shape[0]
    f32 = jnp.float32
    ped_t = ped_obs.reshape(bb, 1, _DIN).swapaxes(1, 2).astype(f32)
    neis_t = neis_obs.reshape(bb, _NN, _DIN).swapaxes(1, 2).astype(f32)
    modes_t = motion_modes.reshape(_K, _DMODE).T.astype(f32)
    mask_col = mask[:, 0, :, None].astype(f32)

    ident256 = jnp.eye(_K, dtype=f32)
    ident100 = jnp.eye(_PK, dtype=f32)
    tri256 = jnp.triu(jnp.ones((_K, _K), f32), 1)
    tri100 = jnp.triu(jnp.ones((_PK, _PK), f32), 1)

    pargs = []
    for name in _PARAM_ORDER:
        w = params[name].astype(f32)
        if w.ndim == 1:
            w = w.reshape(-1, 1)
        pargs.append(w)

    in_specs = [
        pl.BlockSpec((_GRP, _DIN, 1), lambda b: (b, 0, 0)),
        pl.BlockSpec((_GRP, _DIN, _NN), lambda b: (b, 0, 0)),
        pl.BlockSpec((_DMODE, _K), lambda b: (0, 0)),
        pl.BlockSpec((_GRP, _NN, 1), lambda b: (b, 0, 0)),
        pl.BlockSpec((_K, _K), lambda b: (0, 0)),
        pl.BlockSpec((_PK, _PK), lambda b: (0, 0)),
        pl.BlockSpec((_K, _K), lambda b: (0, 0)),
        pl.BlockSpec((_PK, _PK), lambda b: (0, 0)),
    ] + [pl.BlockSpec(w.shape, lambda b, nd=w.ndim: (0,) * nd) for w in pargs]

    out_specs = (
        pl.BlockSpec((_GRP, 2 * _PRED, _NK), lambda b: (b, 0, 0)),
        pl.BlockSpec((_GRP, 1, _PK), lambda b: (b, 0, 0)),
    )
    out_shape = (
        jax.ShapeDtypeStruct((bb, 2 * _PRED, _NK), f32),
        jax.ShapeDtypeStruct((bb, 1, _PK), f32),
    )

    pred_t, sn = pl.pallas_call(
        _body,
        grid=(bb // _GRP,),
        in_specs=in_specs,
        out_specs=out_specs,
        out_shape=out_shape,
        compiler_params=pltpu.CompilerParams(
            dimension_semantics=("parallel",)),
    )(ped_t, neis_t, modes_t, mask_col, ident256, ident100, tri256, tri100,
      *pargs)
    return (pred_t.swapaxes(1, 2), sn.reshape(bb, _PK))
